# Initial kernel scaffold; baseline (speedup 1.0000x reference)
import jax
import jax.numpy as jnp
from jax.experimental import pallas as pl


def kernel(x, Wr, W1, W2, W3):
    raise NotImplementedError("write your pallas kernel here")


# speedup vs baseline: 2.1819x; 2.1819x over previous
import functools

import jax
import jax.numpy as jnp
from jax.experimental import pallas as pl
from jax.experimental.pallas import tpu as pltpu

_TOP_K = 2
_EPAD = 128


def _round_up(v, m):
    return ((v + m - 1) // m) * m


def _router(x_f32, wrt, e_count):
    logits = jnp.dot(x_f32, wrt, preferred_element_type=jnp.float32)
    idx = jax.lax.broadcasted_iota(jnp.int32, logits.shape, 1)
    neg = jnp.float32(-1e30)
    logits = jnp.where(idx < e_count, logits, neg)
    m1 = jnp.max(logits, axis=-1, keepdims=True)
    i1 = jnp.min(jnp.where(logits == m1, idx, e_count), axis=-1, keepdims=True)
    l2 = jnp.where(idx == i1, neg, logits)
    m2 = jnp.max(l2, axis=-1, keepdims=True)
    i2 = jnp.min(jnp.where(l2 == m2, idx, e_count), axis=-1, keepdims=True)
    p = jnp.where(idx < e_count, jnp.exp(logits - m1), 0.0)
    p = p / jnp.sum(p, axis=-1, keepdims=True)
    p1 = jnp.sum(jnp.where(idx == i1, p, 0.0), axis=-1, keepdims=True)
    p2 = jnp.sum(jnp.where(idx == i2, p, 0.0), axis=-1, keepdims=True)
    return i1, i2, p1, p2


def _moe_dense_body(x_ref, wrt_ref, w1t_ref, w3t_ref, w2t_ref, out_ref, *, e_count):
    e = pl.program_id(1)
    x = x_ref[...]
    i1, i2, p1, p2 = _router(x, wrt_ref[...], e_count)
    w = jnp.where(i1 == e, p1, 0.0) + jnp.where(i2 == e, p2, 0.0)

    xb = x.astype(jnp.bfloat16)
    a = jnp.dot(xb, w1t_ref[0], preferred_element_type=jnp.float32)
    b = jnp.dot(xb, w3t_ref[0], preferred_element_type=jnp.float32)
    h = (a * jax.nn.sigmoid(a) * b).astype(jnp.bfloat16)
    y = jnp.dot(h, w2t_ref[0], preferred_element_type=jnp.float32)
    contrib = y * w

    @pl.when(e == 0)
    def _init():
        out_ref[...] = contrib

    @pl.when(e != 0)
    def _acc():
        out_ref[...] += contrib


def kernel(x, Wr, W1, W2, W3):
    B, T, D = x.shape
    E, H, _ = W1.shape
    N = B * T
    HP = _round_up(H, 256)
    BM = min(512, N)

    flat = x.reshape(N, D)
    wrt = jnp.zeros((D, _EPAD), jnp.float32).at[:, :E].set(Wr.T)
    w1t = jnp.pad(W1, ((0, 0), (0, HP - H), (0, 0))).transpose(0, 2, 1).astype(jnp.bfloat16)
    w3t = jnp.pad(W3, ((0, 0), (0, HP - H), (0, 0))).transpose(0, 2, 1).astype(jnp.bfloat16)
    w2t = jnp.pad(W2, ((0, 0), (0, 0), (0, HP - H))).transpose(0, 2, 1).astype(jnp.bfloat16)

    grid = (N // BM, E)
    out = pl.pallas_call(
        functools.partial(_moe_dense_body, e_count=E),
        grid=grid,
        in_specs=[
            pl.BlockSpec((BM, D), lambda m, e: (m, 0)),
            pl.BlockSpec((D, _EPAD), lambda m, e: (0, 0)),
            pl.BlockSpec((1, D, HP), lambda m, e: (e, 0, 0)),
            pl.BlockSpec((1, D, HP), lambda m, e: (e, 0, 0)),
            pl.BlockSpec((1, HP, D), lambda m, e: (e, 0, 0)),
        ],
        out_specs=pl.BlockSpec((BM, D), lambda m, e: (m, 0)),
        out_shape=jax.ShapeDtypeStruct((N, D), jnp.float32),
        compiler_params=pltpu.CompilerParams(
            dimension_semantics=("parallel", "arbitrary"),
        ),
    )(flat, wrt, w1t, w3t, w2t)
    return out.reshape(B, T, D)


# speedup vs baseline: 2.7339x; 1.2530x over previous
import functools

import jax
import jax.numpy as jnp
from jax import lax
from jax.experimental import pallas as pl
from jax.experimental.pallas import tpu as pltpu
from jax.experimental.pallas import tpu_sc as plsc

_TOP_K = 2
_EPAD = 128
_BM = 256


def _round_up(v, m):
    return ((v + m - 1) // m) * m


def _router_math(x_f32, wrt, e_count):
    logits = jnp.dot(x_f32, wrt, preferred_element_type=jnp.float32)
    idx = jax.lax.broadcasted_iota(jnp.int32, logits.shape, 1)
    neg = jnp.float32(-1e30)
    logits = jnp.where(idx < e_count, logits, neg)
    m1 = jnp.max(logits, axis=-1, keepdims=True)
    i1 = jnp.min(jnp.where(logits == m1, idx, e_count), axis=-1, keepdims=True)
    l2 = jnp.where(idx == i1, neg, logits)
    m2 = jnp.max(l2, axis=-1, keepdims=True)
    i2 = jnp.min(jnp.where(l2 == m2, idx, e_count), axis=-1, keepdims=True)
    p = jnp.where(idx < e_count, jnp.exp(logits - m1), 0.0)
    p = p / jnp.sum(p, axis=-1, keepdims=True)
    p1 = jnp.sum(jnp.where(idx == i1, p, 0.0), axis=-1, keepdims=True)
    p2 = jnp.sum(jnp.where(idx == i2, p, 0.0), axis=-1, keepdims=True)
    return i1, i2, p1, p2


def _router_body(x_ref, wrt_ref, i1_ref, i2_ref, p1_ref, p2_ref, *, e_count):
    i1, i2, p1, p2 = _router_math(x_ref[...], wrt_ref[...], e_count)
    i1_ref[...] = i1
    i2_ref[...] = i2
    p1_ref[...] = p1
    p2_ref[...] = p2


def _run_router(flat, Wr, E):
    N, D = flat.shape
    BR = min(1024, N)
    wrt = jnp.zeros((D, _EPAD), jnp.float32).at[:, :E].set(Wr.T)
    o = jax.ShapeDtypeStruct((N, 1), jnp.int32)
    of = jax.ShapeDtypeStruct((N, 1), jnp.float32)
    return pl.pallas_call(
        functools.partial(_router_body, e_count=E),
        grid=(N // BR,),
        in_specs=[
            pl.BlockSpec((BR, D), lambda m: (m, 0)),
            pl.BlockSpec((D, _EPAD), lambda m: (0, 0)),
        ],
        out_specs=[pl.BlockSpec((BR, 1), lambda m: (m, 0))] * 4,
        out_shape=[o, o, of, of],
        compiler_params=pltpu.CompilerParams(
            dimension_semantics=("parallel",),
        ),
    )(flat, wrt)


def _make_sc_gather(N, D, R, n_rows):
    NW = 32
    per_w = n_rows // NW
    CH = 64
    n_chunks = per_w // CH
    mesh = plsc.VectorSubcoreMesh(core_axis_name="c", subcore_axis_name="s")

    @functools.partial(
        pl.kernel,
        mesh=mesh,
        out_type=jax.ShapeDtypeStruct((n_rows, D), jnp.float32),
        scratch_types=[
            pltpu.VMEM((CH,), jnp.int32),
            pltpu.VMEM((CH, D), jnp.float32),
            pltpu.SemaphoreType.DMA,
        ],
    )
    def gather_k(flat_hbm, rowtok_hbm, out_hbm, idx_v, rows_v, sem):
        wid = lax.axis_index("s") * 2 + lax.axis_index("c")
        base = wid * per_w
        for c in range(n_chunks):
            off = base + c * CH
            pltpu.sync_copy(rowtok_hbm.at[pl.ds(off, CH)], idx_v)
            pltpu.async_copy(flat_hbm.at[idx_v], rows_v, sem).wait()
            pltpu.sync_copy(rows_v, out_hbm.at[pl.ds(off, CH)])

    return gather_k


def _gemm_body(be_ref, xd_ref, gate_ref, w1t_ref, w3t_ref, w2t_ref, out_ref):
    xb = xd_ref[...].astype(jnp.bfloat16)
    a = jnp.dot(xb, w1t_ref[0], preferred_element_type=jnp.float32)
    b = jnp.dot(xb, w3t_ref[0], preferred_element_type=jnp.float32)
    h = (a * jax.nn.sigmoid(a) * b).astype(jnp.bfloat16)
    y = jnp.dot(h, w2t_ref[0], preferred_element_type=jnp.float32)
    out_ref[...] = y * gate_ref[...]


def _run_gemm(xd, gates, w1t, w3t, w2t, block_expert, G, D, HP):
    n_rows = G * _BM
    grid_spec = pltpu.PrefetchScalarGridSpec(
        num_scalar_prefetch=1,
        grid=(G,),
        in_specs=[
            pl.BlockSpec((_BM, D), lambda g, be: (g, 0)),
            pl.BlockSpec((_BM, 1), lambda g, be: (g, 0)),
            pl.BlockSpec((1, D, HP), lambda g, be: (be[g], 0, 0)),
            pl.BlockSpec((1, D, HP), lambda g, be: (be[g], 0, 0)),
            pl.BlockSpec((1, HP, D), lambda g, be: (be[g], 0, 0)),
        ],
        out_specs=pl.BlockSpec((_BM, D), lambda g, be: (g, 0)),
    )
    return pl.pallas_call(
        _gemm_body,
        grid_spec=grid_spec,
        out_shape=jax.ShapeDtypeStruct((n_rows, D), jnp.float32),
        compiler_params=pltpu.CompilerParams(
            dimension_semantics=("arbitrary",),
        ),
    )(block_expert, xd, gates, w1t, w3t, w2t)


def _make_sc_combine(n_rows, N, D):
    NW = 32
    per_w = N // NW
    CH = 32
    n_chunks = per_w // CH
    mesh = plsc.VectorSubcoreMesh(core_axis_name="c", subcore_axis_name="s")

    @functools.partial(
        pl.kernel,
        mesh=mesh,
        out_type=jax.ShapeDtypeStruct((2, N, D), jnp.float32),
        scratch_types=[
            pltpu.VMEM((CH,), jnp.int32),
            pltpu.VMEM((CH, D), jnp.float32),
            pltpu.SemaphoreType.DMA,
        ],
    )
    def combine_k(y_hbm, pos0_hbm, pos1_hbm, out_hbm, idx_v, rows_v, sem):
        wid = lax.axis_index("s") * 2 + lax.axis_index("c")
        base = wid * per_w
        for c in range(n_chunks):
            off = base + c * CH
            pltpu.sync_copy(pos0_hbm.at[pl.ds(off, CH)], idx_v)
            pltpu.async_copy(y_hbm.at[idx_v], rows_v, sem).wait()
            pltpu.sync_copy(rows_v, out_hbm.at[0, pl.ds(off, CH)])
            pltpu.sync_copy(pos1_hbm.at[pl.ds(off, CH)], idx_v)
            pltpu.async_copy(y_hbm.at[idx_v], rows_v, sem).wait()
            pltpu.sync_copy(rows_v, out_hbm.at[1, pl.ds(off, CH)])

    return combine_k


def _add_body(y01_ref, out_ref):
    out_ref[...] = y01_ref[0] + y01_ref[1]


def _run_add(y01, N, D):
    BR = min(1024, N)
    return pl.pallas_call(
        _add_body,
        grid=(N // BR,),
        in_specs=[pl.BlockSpec((2, BR, D), lambda m: (0, m, 0))],
        out_specs=pl.BlockSpec((BR, D), lambda m: (m, 0)),
        out_shape=jax.ShapeDtypeStruct((N, D), jnp.float32),
        compiler_params=pltpu.CompilerParams(
            dimension_semantics=("parallel",),
        ),
    )(y01)


def kernel(x, Wr, W1, W2, W3):
    B, T, D = x.shape
    E, H, _ = W1.shape
    N = B * T
    NK = N * _TOP_K
    HP = _round_up(H, 256)
    G = NK // _BM + E
    n_rows = G * _BM

    flat = x.reshape(N, D)

    i1, i2, p1, p2 = _run_router(flat, Wr, E)

    te = jnp.concatenate([i1, i2], axis=1).reshape(-1)
    wv = jnp.concatenate([p1, p2], axis=1).reshape(-1)
    onehot = (te[:, None] == jnp.arange(E)[None, :]).astype(jnp.int32)
    cnt_incl = jnp.cumsum(onehot, axis=0)
    rank = jnp.take_along_axis(cnt_incl, te[:, None], axis=1)[:, 0] - 1
    count = cnt_incl[-1]
    blocks = -(-count // _BM)
    bstart = jnp.concatenate(
        [jnp.zeros((1,), blocks.dtype), jnp.cumsum(blocks)])[:E]
    pos = (bstart[te] * _BM + rank).astype(jnp.int32)
    row_token = jnp.zeros((n_rows,), jnp.int32).at[pos].set(
        jnp.arange(NK, dtype=jnp.int32) // _TOP_K)
    row_gate = jnp.zeros((n_rows, 1), jnp.float32).at[pos, 0].set(wv)
    block_expert = (jnp.sum(
        jnp.arange(G)[:, None] >= bstart[None, :], axis=1) - 1).astype(jnp.int32)
    posm = pos.reshape(N, _TOP_K)
    pos0 = posm[:, 0]
    pos1 = posm[:, 1]

    xd = _make_sc_gather(N, D, None, n_rows)(flat, row_token)

    w1t = jnp.pad(W1, ((0, 0), (0, HP - H), (0, 0))).transpose(0, 2, 1).astype(jnp.bfloat16)
    w3t = jnp.pad(W3, ((0, 0), (0, HP - H), (0, 0))).transpose(0, 2, 1).astype(jnp.bfloat16)
    w2t = jnp.pad(W2, ((0, 0), (0, 0), (0, HP - H))).transpose(0, 2, 1).astype(jnp.bfloat16)
    y = _run_gemm(xd, row_gate, w1t, w3t, w2t, block_expert, G, D, HP)

    y01 = _make_sc_combine(n_rows, N, D)(y, pos0, pos1)
    out = _run_add(y01, N, D)
    return out.reshape(B, T, D)


# speedup vs baseline: 3.3874x; 1.2391x over previous
import functools

import jax
import jax.numpy as jnp
from jax import lax
from jax.experimental import pallas as pl
from jax.experimental.pallas import tpu as pltpu
from jax.experimental.pallas import tpu_sc as plsc

_TOP_K = 2
_EPAD = 128
_BM = 256
_NW = 32
_CH = 64
_BR = 128


def _router_body(x_ref, wrt_ref, p1_ref, p2_ref, t1_ref, t2_ref, *, e_count):
    logits = jnp.dot(x_ref[...], wrt_ref[...],
                     preferred_element_type=jnp.float32)
    idx = jax.lax.broadcasted_iota(jnp.int32, logits.shape, 1)
    neg = jnp.float32(-1e30)
    logits = jnp.where(idx < e_count, logits, neg)
    m1 = jnp.max(logits, axis=-1, keepdims=True)
    i1 = jnp.min(jnp.where(logits == m1, idx, e_count), axis=-1, keepdims=True)
    l2 = jnp.where(idx == i1, neg, logits)
    m2 = jnp.max(l2, axis=-1, keepdims=True)
    i2 = jnp.min(jnp.where(l2 == m2, idx, e_count), axis=-1, keepdims=True)
    p = jnp.where(idx < e_count, jnp.exp(logits - m1), 0.0)
    p = p / jnp.sum(p, axis=-1, keepdims=True)
    p1_ref[...] = jnp.sum(jnp.where(idx == i1, p, 0.0), axis=-1, keepdims=True)
    p2_ref[...] = jnp.sum(jnp.where(idx == i2, p, 0.0), axis=-1, keepdims=True)
    t1_ref[...] = jnp.transpose(i1.astype(jnp.float32))[None]
    t2_ref[...] = jnp.transpose(i2.astype(jnp.float32))[None]


def _run_router(flat, Wr, E):
    N, D = flat.shape
    wrt = jnp.zeros((D, _EPAD), jnp.float32).at[:, :E].set(Wr.T)
    nb = N // _BR
    of = jax.ShapeDtypeStruct((N, 1), jnp.float32)
    ot = jax.ShapeDtypeStruct((nb, 1, _BR), jnp.float32)
    return pl.pallas_call(
        functools.partial(_router_body, e_count=E),
        grid=(nb,),
        in_specs=[
            pl.BlockSpec((_BR, D), lambda m: (m, 0)),
            pl.BlockSpec((D, _EPAD), lambda m: (0, 0)),
        ],
        out_specs=[
            pl.BlockSpec((_BR, 1), lambda m: (m, 0)),
            pl.BlockSpec((_BR, 1), lambda m: (m, 0)),
            pl.BlockSpec((1, 1, _BR), lambda m: (m, 0, 0)),
            pl.BlockSpec((1, 1, _BR), lambda m: (m, 0, 0)),
        ],
        out_shape=[of, of, ot, ot],
        compiler_params=pltpu.CompilerParams(
            dimension_semantics=("parallel",),
        ),
    )(flat, wrt)


def _meta_body(t1_ref, t2_ref, pos_ref, be_ref, *, e_count, g_count):
    te = jnp.concatenate([t1_ref[...][:, 0, :], t2_ref[...][:, 0, :]], axis=0)
    ra = te.shape[0]
    tri_l = (jax.lax.broadcasted_iota(jnp.int32, (_BR, _BR), 0)
             <= jax.lax.broadcasted_iota(jnp.int32, (_BR, _BR), 1)
             ).astype(jnp.float32)
    tri_r = (jax.lax.broadcasted_iota(jnp.int32, (ra, ra), 1)
             < jax.lax.broadcasted_iota(jnp.int32, (ra, ra), 0)
             ).astype(jnp.float32)
    pos = jnp.zeros(te.shape, jnp.float32)
    bstart = jnp.zeros((1, 1), jnp.float32)
    iota_g = jax.lax.broadcasted_iota(jnp.int32, (1, _EPAD), 1)
    be = jnp.zeros((1, _EPAD), jnp.int32)
    for e in range(e_count):
        m = (te == jnp.float32(e)).astype(jnp.float32)
        p = jnp.dot(m, tri_l, preferred_element_type=jnp.float32)
        rowtot = p[:, _BR - 1:_BR]
        rowpre = jnp.dot(tri_r, rowtot, preferred_element_type=jnp.float32)
        incl = p + rowpre
        cnt = incl[ra - 1:ra, _BR - 1:_BR]
        pos = jnp.where(m > 0.0, bstart * _BM + incl - 1.0, pos)
        nblk = jnp.floor((cnt + jnp.float32(_BM - 1)) * (1.0 / _BM))
        be = be + jnp.where(
            iota_g >= bstart.astype(jnp.int32), 1, 0)
        bstart = bstart + nblk
    pos_ref[...] = pos.astype(jnp.int32)
    be_ref[...] = be - 1


def _run_meta(t1, t2, E, G):
    nb = t1.shape[0]
    return pl.pallas_call(
        functools.partial(_meta_body, e_count=E, g_count=G),
        out_shape=[
            jax.ShapeDtypeStruct((2 * nb, _BR), jnp.int32),
            jax.ShapeDtypeStruct((1, _EPAD), jnp.int32),
        ],
    )(t1, t2)


def _make_sc_dispatch(N, D, n_rows):
    per_w = N // _NW
    n_chunks = per_w // _CH
    mesh = plsc.VectorSubcoreMesh(core_axis_name="c", subcore_axis_name="s")

    @functools.partial(
        pl.kernel,
        mesh=mesh,
        out_type=jax.ShapeDtypeStruct((n_rows, D), jnp.float32),
        scratch_types=[
            pltpu.VMEM((_CH, D), jnp.float32),
            pltpu.VMEM((_CH,), jnp.int32),
            pltpu.VMEM((_CH,), jnp.int32),
            pltpu.SemaphoreType.DMA,
            pltpu.SemaphoreType.DMA,
        ],
    )
    def dispatch_k(flat_hbm, pos0_hbm, pos1_hbm, xd_hbm, rows_v, idx0_v,
                   idx1_v, sem0, sem1):
        wid = lax.axis_index("s") * 2 + lax.axis_index("c")
        base = wid * per_w
        for c in range(n_chunks):
            pltpu.sync_copy(pos0_hbm.at[wid, c], idx0_v)
            pltpu.sync_copy(pos1_hbm.at[wid, c], idx1_v)
            pltpu.sync_copy(flat_hbm.at[pl.ds(base + c * _CH, _CH)], rows_v)
            c0 = pltpu.async_copy(rows_v, xd_hbm.at[idx0_v], sem0)
            c1 = pltpu.async_copy(rows_v, xd_hbm.at[idx1_v], sem1)
            c0.wait()
            c1.wait()

    return dispatch_k


def _gemm_body(be_ref, xd_ref, w1_ref, w3_ref, w2_ref, out_ref):
    xb = xd_ref[...].astype(jnp.bfloat16)
    dn = (((1,), (1,)), ((), ()))
    a = lax.dot_general(xb, w1_ref[0], dn, preferred_element_type=jnp.float32)
    b = lax.dot_general(xb, w3_ref[0], dn, preferred_element_type=jnp.float32)
    h = (a * jax.nn.sigmoid(a) * b).astype(jnp.bfloat16)
    y = lax.dot_general(h, w2_ref[0], dn, preferred_element_type=jnp.float32)
    out_ref[...] = y


def _run_gemm(xd, w1b, w3b, w2b, block_expert, G, D, H):
    n_rows = G * _BM
    grid_spec = pltpu.PrefetchScalarGridSpec(
        num_scalar_prefetch=1,
        grid=(G,),
        in_specs=[
            pl.BlockSpec((_BM, D), lambda g, be: (g, 0)),
            pl.BlockSpec((1, H, D), lambda g, be: (be[g], 0, 0)),
            pl.BlockSpec((1, H, D), lambda g, be: (be[g], 0, 0)),
            pl.BlockSpec((1, D, H), lambda g, be: (be[g], 0, 0)),
        ],
        out_specs=pl.BlockSpec((_BM, D), lambda g, be: (g, 0)),
    )
    return pl.pallas_call(
        _gemm_body,
        grid_spec=grid_spec,
        out_shape=jax.ShapeDtypeStruct((n_rows, D), jnp.float32),
        compiler_params=pltpu.CompilerParams(
            dimension_semantics=("arbitrary",),
            vmem_limit_bytes=100 * 1024 * 1024,
        ),
    )(block_expert, xd, w1b, w3b, w2b)


def _make_sc_combine(n_rows, N, D):
    per_w = N // _NW
    n_chunks = per_w // _CH
    mesh = plsc.VectorSubcoreMesh(core_axis_name="c", subcore_axis_name="s")

    @functools.partial(
        pl.kernel,
        mesh=mesh,
        out_type=jax.ShapeDtypeStruct((2, N, D), jnp.float32),
        scratch_types=[
            pltpu.VMEM((_CH,), jnp.int32),
            pltpu.VMEM((_CH, D), jnp.float32),
            pltpu.SemaphoreType.DMA,
        ],
    )
    def combine_k(y_hbm, pos0_hbm, pos1_hbm, out_hbm, idx_v, rows_v, sem):
        wid = lax.axis_index("s") * 2 + lax.axis_index("c")
        base = wid * per_w
        for c in range(n_chunks):
            off = base + c * _CH
            pltpu.sync_copy(pos0_hbm.at[pl.ds(off, _CH)], idx_v)
            pltpu.async_copy(y_hbm.at[idx_v], rows_v, sem).wait()
            pltpu.sync_copy(rows_v, out_hbm.at[0, pl.ds(off, _CH)])
            pltpu.sync_copy(pos1_hbm.at[pl.ds(off, _CH)], idx_v)
            pltpu.async_copy(y_hbm.at[idx_v], rows_v, sem).wait()
            pltpu.sync_copy(rows_v, out_hbm.at[1, pl.ds(off, _CH)])

    return combine_k


def _add_body(y01_ref, p1_ref, p2_ref, out_ref):
    out_ref[...] = y01_ref[0] * p1_ref[...] + y01_ref[1] * p2_ref[...]


def _run_add(y01, p1, p2, N, D):
    BR = min(1024, N)
    return pl.pallas_call(
        _add_body,
        grid=(N // BR,),
        in_specs=[
            pl.BlockSpec((2, BR, D), lambda m: (0, m, 0)),
            pl.BlockSpec((BR, 1), lambda m: (m, 0)),
            pl.BlockSpec((BR, 1), lambda m: (m, 0)),
        ],
        out_specs=pl.BlockSpec((BR, D), lambda m: (m, 0)),
        out_shape=jax.ShapeDtypeStruct((N, D), jnp.float32),
        compiler_params=pltpu.CompilerParams(
            dimension_semantics=("parallel",),
        ),
    )(y01, p1, p2)


def kernel(x, Wr, W1, W2, W3):
    B, T, D = x.shape
    E, H, _ = W1.shape
    N = B * T
    NK = N * _TOP_K
    G = NK // _BM + E
    n_rows = G * _BM

    flat = x.reshape(N, D)

    p1, p2, t1, t2 = _run_router(flat, Wr, E)
    pos_cat, be_pad = _run_meta(t1, t2, E, G)

    pos_flat = pos_cat.reshape(-1)
    pos0 = pos_flat[:N]
    pos1 = pos_flat[N:]
    pos0_3d = pos0.reshape(_NW, N // _NW // _CH, _CH)
    pos1_3d = pos1.reshape(_NW, N // _NW // _CH, _CH)
    block_expert = be_pad[0, :G]

    xd = _make_sc_dispatch(N, D, n_rows)(flat, pos0_3d, pos1_3d)

    w1b = W1.astype(jnp.bfloat16)
    w3b = W3.astype(jnp.bfloat16)
    w2b = W2.astype(jnp.bfloat16)
    y = _run_gemm(xd, w1b, w3b, w2b, block_expert, G, D, H)

    y01 = _make_sc_combine(n_rows, N, D)(y, pos0, pos1)
    out = _run_add(y01, p1, p2, N, D)
    return out.reshape(B, T, D)


# speedup vs baseline: 3.8922x; 1.1490x over previous
import functools

import jax
import jax.numpy as jnp
from jax import lax
from jax.experimental import pallas as pl
from jax.experimental.pallas import tpu as pltpu
from jax.experimental.pallas import tpu_sc as plsc

_TOP_K = 2
_EPAD = 128
_BM = 256
_NW = 32
_CH = 64
_BR = 128


def _router_body(x_ref, wrt_ref, p1_ref, p2_ref, t1_ref, t2_ref, *, e_count):
    logits = jnp.dot(x_ref[...], wrt_ref[...],
                     preferred_element_type=jnp.float32)
    idx = jax.lax.broadcasted_iota(jnp.int32, logits.shape, 1)
    neg = jnp.float32(-1e30)
    logits = jnp.where(idx < e_count, logits, neg)
    m1 = jnp.max(logits, axis=-1, keepdims=True)
    i1 = jnp.min(jnp.where(logits == m1, idx, e_count), axis=-1, keepdims=True)
    l2 = jnp.where(idx == i1, neg, logits)
    m2 = jnp.max(l2, axis=-1, keepdims=True)
    i2 = jnp.min(jnp.where(l2 == m2, idx, e_count), axis=-1, keepdims=True)
    p = jnp.where(idx < e_count, jnp.exp(logits - m1), 0.0)
    p = p / jnp.sum(p, axis=-1, keepdims=True)
    p1_ref[...] = jnp.sum(jnp.where(idx == i1, p, 0.0), axis=-1, keepdims=True)
    p2_ref[...] = jnp.sum(jnp.where(idx == i2, p, 0.0), axis=-1, keepdims=True)
    t1_ref[...] = jnp.transpose(i1.astype(jnp.float32))[None]
    t2_ref[...] = jnp.transpose(i2.astype(jnp.float32))[None]


def _run_router(flat, Wr, E):
    N, D = flat.shape
    wrt = jnp.zeros((D, _EPAD), jnp.float32).at[:, :E].set(Wr.T)
    nb = N // _BR
    of = jax.ShapeDtypeStruct((N, 1), jnp.float32)
    ot = jax.ShapeDtypeStruct((nb, 1, _BR), jnp.float32)
    return pl.pallas_call(
        functools.partial(_router_body, e_count=E),
        grid=(nb,),
        in_specs=[
            pl.BlockSpec((_BR, D), lambda m: (m, 0)),
            pl.BlockSpec((D, _EPAD), lambda m: (0, 0)),
        ],
        out_specs=[
            pl.BlockSpec((_BR, 1), lambda m: (m, 0)),
            pl.BlockSpec((_BR, 1), lambda m: (m, 0)),
            pl.BlockSpec((1, 1, _BR), lambda m: (m, 0, 0)),
            pl.BlockSpec((1, 1, _BR), lambda m: (m, 0, 0)),
        ],
        out_shape=[of, of, ot, ot],
        compiler_params=pltpu.CompilerParams(
            dimension_semantics=("parallel",),
        ),
    )(flat, wrt)


def _meta_body(t1_ref, t2_ref, pos_ref, be_ref, *, e_count, g_count):
    te = jnp.concatenate([t1_ref[...][:, 0, :], t2_ref[...][:, 0, :]], axis=0)
    ra = te.shape[0]
    tri_l = (jax.lax.broadcasted_iota(jnp.int32, (_BR, _BR), 0)
             <= jax.lax.broadcasted_iota(jnp.int32, (_BR, _BR), 1)
             ).astype(jnp.float32)
    tri_r = (jax.lax.broadcasted_iota(jnp.int32, (ra, ra), 1)
             < jax.lax.broadcasted_iota(jnp.int32, (ra, ra), 0)
             ).astype(jnp.float32)
    pos = jnp.zeros(te.shape, jnp.float32)
    bstart = jnp.zeros((1, 1), jnp.float32)
    iota_g = jax.lax.broadcasted_iota(jnp.int32, (1, _EPAD), 1)
    be = jnp.zeros((1, _EPAD), jnp.int32)
    for e in range(e_count):
        m = (te == jnp.float32(e)).astype(jnp.float32)
        p = jnp.dot(m, tri_l, preferred_element_type=jnp.float32)
        rowtot = p[:, _BR - 1:_BR]
        rowpre = jnp.dot(tri_r, rowtot, preferred_element_type=jnp.float32)
        incl = p + rowpre
        cnt = incl[ra - 1:ra, _BR - 1:_BR]
        pos = jnp.where(m > 0.0, bstart * _BM + incl - 1.0, pos)
        nblk = jnp.floor((cnt + jnp.float32(_BM - 1)) * (1.0 / _BM))
        be = be + jnp.where(
            iota_g >= bstart.astype(jnp.int32), 1, 0)
        bstart = bstart + nblk
    pos_ref[...] = pos.astype(jnp.int32)
    be_ref[...] = be - 1


def _run_meta(t1, t2, E, G):
    nb = t1.shape[0]
    return pl.pallas_call(
        functools.partial(_meta_body, e_count=E, g_count=G),
        out_shape=[
            jax.ShapeDtypeStruct((2 * nb, _BR), jnp.int32),
            jax.ShapeDtypeStruct((1, _EPAD), jnp.int32),
        ],
    )(t1, t2)


def _make_sc_dispatch(N, D, n_rows):
    per_w = N // _NW
    n_chunks = per_w // _CH
    mesh = plsc.VectorSubcoreMesh(core_axis_name="c", subcore_axis_name="s")

    @functools.partial(
        pl.kernel,
        mesh=mesh,
        out_type=jax.ShapeDtypeStruct((n_rows, D), jnp.float32),
        scratch_types=[
            pltpu.VMEM((_CH, D), jnp.float32),
            pltpu.VMEM((_CH,), jnp.int32),
            pltpu.VMEM((_CH,), jnp.int32),
            pltpu.SemaphoreType.DMA,
            pltpu.SemaphoreType.DMA,
        ],
    )
    def dispatch_k(flat_hbm, pos0_hbm, pos1_hbm, xd_hbm, rows_v, idx0_v,
                   idx1_v, sem0, sem1):
        wid = lax.axis_index("s") * 2 + lax.axis_index("c")
        base = wid * per_w
        for c in range(n_chunks):
            pltpu.sync_copy(pos0_hbm.at[wid, c], idx0_v)
            pltpu.sync_copy(pos1_hbm.at[wid, c], idx1_v)
            pltpu.sync_copy(flat_hbm.at[pl.ds(base + c * _CH, _CH)], rows_v)
            c0 = pltpu.async_copy(rows_v, xd_hbm.at[idx0_v], sem0)
            c1 = pltpu.async_copy(rows_v, xd_hbm.at[idx1_v], sem1)
            c0.wait()
            c1.wait()

    return dispatch_k


def _gemm_body(be_ref, xd_ref, w1_ref, w3_ref, w2_ref, out_ref):
    xb = xd_ref[...].astype(jnp.bfloat16)
    a = jnp.dot(xb, w1_ref[0], preferred_element_type=jnp.float32)
    b = jnp.dot(xb, w3_ref[0], preferred_element_type=jnp.float32)
    h = (a * jax.nn.sigmoid(a) * b).astype(jnp.bfloat16)
    y = jnp.dot(h, w2_ref[0], preferred_element_type=jnp.float32)
    out_ref[...] = y


def _run_gemm(xd, w1b, w3b, w2b, block_expert, G, D, H):
    n_rows = G * _BM
    grid_spec = pltpu.PrefetchScalarGridSpec(
        num_scalar_prefetch=1,
        grid=(G,),
        in_specs=[
            pl.BlockSpec((_BM, D), lambda g, be: (g, 0)),
            pl.BlockSpec((1, D, H), lambda g, be: (be[g], 0, 0)),
            pl.BlockSpec((1, D, H), lambda g, be: (be[g], 0, 0)),
            pl.BlockSpec((1, H, D), lambda g, be: (be[g], 0, 0)),
        ],
        out_specs=pl.BlockSpec((_BM, D), lambda g, be: (g, 0)),
    )
    return pl.pallas_call(
        _gemm_body,
        grid_spec=grid_spec,
        out_shape=jax.ShapeDtypeStruct((n_rows, D), jnp.float32),
        compiler_params=pltpu.CompilerParams(
            dimension_semantics=("arbitrary",),
        ),
    )(block_expert, xd, w1b, w3b, w2b)


def _make_sc_combine(n_rows, N, D):
    per_w = N // _NW
    n_chunks = per_w // _CH
    mesh = plsc.VectorSubcoreMesh(core_axis_name="c", subcore_axis_name="s")

    @functools.partial(
        pl.kernel,
        mesh=mesh,
        out_type=jax.ShapeDtypeStruct((2, N, D), jnp.float32),
        scratch_types=[
            pltpu.VMEM((_CH,), jnp.int32),
            pltpu.VMEM((_CH, D), jnp.float32),
            pltpu.SemaphoreType.DMA,
        ],
    )
    def combine_k(y_hbm, pos0_hbm, pos1_hbm, out_hbm, idx_v, rows_v, sem):
        wid = lax.axis_index("s") * 2 + lax.axis_index("c")
        base = wid * per_w
        for c in range(n_chunks):
            off = base + c * _CH
            pltpu.sync_copy(pos0_hbm.at[pl.ds(off, _CH)], idx_v)
            pltpu.async_copy(y_hbm.at[idx_v], rows_v, sem).wait()
            pltpu.sync_copy(rows_v, out_hbm.at[0, pl.ds(off, _CH)])
            pltpu.sync_copy(pos1_hbm.at[pl.ds(off, _CH)], idx_v)
            pltpu.async_copy(y_hbm.at[idx_v], rows_v, sem).wait()
            pltpu.sync_copy(rows_v, out_hbm.at[1, pl.ds(off, _CH)])

    return combine_k


def _add_body(y01_ref, p1_ref, p2_ref, out_ref):
    out_ref[...] = y01_ref[0] * p1_ref[...] + y01_ref[1] * p2_ref[...]


def _run_add(y01, p1, p2, N, D):
    BR = min(1024, N)
    return pl.pallas_call(
        _add_body,
        grid=(N // BR,),
        in_specs=[
            pl.BlockSpec((2, BR, D), lambda m: (0, m, 0)),
            pl.BlockSpec((BR, 1), lambda m: (m, 0)),
            pl.BlockSpec((BR, 1), lambda m: (m, 0)),
        ],
        out_specs=pl.BlockSpec((BR, D), lambda m: (m, 0)),
        out_shape=jax.ShapeDtypeStruct((N, D), jnp.float32),
        compiler_params=pltpu.CompilerParams(
            dimension_semantics=("parallel",),
        ),
    )(y01, p1, p2)


def kernel(x, Wr, W1, W2, W3):
    B, T, D = x.shape
    E, H, _ = W1.shape
    N = B * T
    NK = N * _TOP_K
    G = NK // _BM + E
    n_rows = G * _BM

    flat = x.reshape(N, D)

    p1, p2, t1, t2 = _run_router(flat, Wr, E)
    pos_cat, be_pad = _run_meta(t1, t2, E, G)

    pos_flat = pos_cat.reshape(-1)
    pos0 = pos_flat[:N]
    pos1 = pos_flat[N:]
    pos0_3d = pos0.reshape(_NW, N // _NW // _CH, _CH)
    pos1_3d = pos1.reshape(_NW, N // _NW // _CH, _CH)
    block_expert = be_pad[0, :G]

    xd = _make_sc_dispatch(N, D, n_rows)(flat, pos0_3d, pos1_3d)

    w1b = W1.astype(jnp.bfloat16).transpose(0, 2, 1)
    w3b = W3.astype(jnp.bfloat16).transpose(0, 2, 1)
    w2b = W2.astype(jnp.bfloat16).transpose(0, 2, 1)
    y = _run_gemm(xd, w1b, w3b, w2b, block_expert, G, D, H)

    y01 = _make_sc_combine(n_rows, N, D)(y, pos0, pos1)
    out = _run_add(y01, p1, p2, N, D)
    return out.reshape(B, T, D)
